# baseline (speedup 1.0000x reference)
import functools

import jax
import jax.numpy as jnp
from jax import lax
from jax.experimental import pallas as pl
from jax.experimental.pallas import tpu as pltpu
from jax.experimental.pallas import tpu_sc as plsc

_V, _D, _H, _O = 1000000, 64, 128, 10
_B, _L = 4096, 200

_NC, _NS = 2, 16
_NW = _NC * _NS
_SPW = _B // _NW
_CA, _CB = 96, 104
_GRP = 4
_NG = _SPW // _GRP

_WIN = 384
_NWIN = _V // _WIN
_TAIL = _V - _NWIN * _WIN
_BASE_WIN = _NWIN // _NW
_EXTRA = _NWIN - _BASE_WIN * _NW


def _tr_body(tab_hbm, dimidx_hbm, tail_hbm, out_hbm,
             didx_v, in0, in1, tr0, tr1, sem0, sem1, osem0, osem1):
    wid = lax.axis_index("s") * _NC + lax.axis_index("c")
    nwin = _BASE_WIN + jnp.where(wid < _EXTRA, 1, 0)
    iota = lax.iota(jnp.int32, 16)
    ib = [iota * _D + 16 * j * _D for j in range(_WIN // 16)]

    pltpu.sync_copy(dimidx_hbm, didx_v)

    ins = (in0, in1)
    sems = (sem0, sem1)
    trs = (tr0, tr1)
    osems = (osem0, osem1)

    def win_of(i):
        return wid + _NW * i

    def start_in(i, buf, sem):
        src = tab_hbm.at[didx_v, pl.ds(win_of(i) * _WIN, _WIN)]
        pltpu.make_async_copy(src, buf, sem).start()

    def wait_in(buf, sem):
        pltpu.make_async_copy(tab_hbm.at[didx_v, pl.ds(0, _WIN)],
                              buf, sem).wait()

    def wait_out(tr, osem):
        pltpu.make_async_copy(tr, out_hbm.at[pl.ds(0, _WIN * _D)], osem).wait()

    def transpose_win(buf, tr):
        def d_body(d4, c):
            for u in range(4):
                d = 4 * d4 + u
                vs = [buf[d, pl.ds(16 * j, 16)] for j in range(_WIN // 16)]
                db = jnp.full((16,), d, jnp.int32)
                for j in range(_WIN // 16):
                    plsc.store_scatter(tr, [ib[j] + db], vs[j])
            return c
        lax.fori_loop(0, _D // 4, d_body, 0)

    for b in range(2):
        start_in(b, ins[b], sems[b])

    def ring_body(g, carry):
        for b in range(2):
            i = 2 * g + b
            buf, sem = ins[b], sems[b]

            @pl.when(i < nwin)
            def _():
                wait_in(buf, sem)
                tr, osem = trs[b], osems[b]

                @pl.when(i >= 2)
                def _():
                    wait_out(tr, osem)

                transpose_win(buf, tr)
                pltpu.make_async_copy(
                    tr, out_hbm.at[pl.ds(win_of(i) * _WIN * _D, _WIN * _D)],
                    osem).start()

                @pl.when(i + 2 < nwin)
                def _():
                    start_in(i + 2, buf, sem)
        return carry

    lax.fori_loop(0, (_BASE_WIN + 1 + 1) // 2, ring_body, 0)

    wait_out(trs[0], osems[0])
    wait_out(trs[1], osems[1])

    @pl.when(wid == _NW - 1)
    def _():
        pltpu.sync_copy(tail_hbm, tr0.at[pl.ds(0, _TAIL * _D)])
        pltpu.sync_copy(tr0.at[pl.ds(0, _TAIL * _D)],
                        out_hbm.at[pl.ds(_NWIN * _WIN * _D, _TAIL * _D)])


def _accum(buf, nrows, acc):
    def rbody(r, carry):
        a0, a1, a2, a3 = carry
        for u in range(4):
            row = 4 * r + u
            a0 = a0 + buf[row, pl.ds(0, 16)]
            a1 = a1 + buf[row, pl.ds(16, 16)]
            a2 = a2 + buf[row, pl.ds(32, 16)]
            a3 = a3 + buf[row, pl.ds(48, 16)]
        return (a0, a1, a2, a3)
    return lax.fori_loop(0, nrows // 4, rbody, acc)


def _pool_body(ids_hbm, table_hbm, out_hbm, idx_v, out_v, *bufsem):
    bufs = bufsem[:2 * _GRP]
    sems = bufsem[2 * _GRP:]
    wid = lax.axis_index("s") * _NC + lax.axis_index("c")
    pltpu.sync_copy(ids_hbm.at[wid], idx_v)

    def start(s, j, buf, sem):
        if j == 0:
            src = table_hbm.at[idx_v.at[pl.ds(_L * s, _CA)]]
        else:
            src = table_hbm.at[idx_v.at[pl.ds(_L * s + _CA, _CB)]]
        pltpu.make_async_copy(src, buf, sem).start()

    def wait(buf, sem):
        pltpu.make_async_copy(table_hbm.at[idx_v.at[pl.ds(0, buf.shape[0])]],
                              buf, sem).wait()

    for j in range(_GRP):
        start(j, 0, bufs[2 * j], sems[2 * j])
        start(j, 1, bufs[2 * j + 1], sems[2 * j + 1])

    zero = jnp.zeros((16,), jnp.float32)

    def group_body(g, carry):
        for j in range(_GRP):
            s = _GRP * g + j
            wait(bufs[2 * j], sems[2 * j])
            acc = _accum(bufs[2 * j], _CA, (zero, zero, zero, zero))

            @pl.when(g < _NG - 1)
            def _():
                start(s + _GRP, 0, bufs[2 * j], sems[2 * j])

            wait(bufs[2 * j + 1], sems[2 * j + 1])
            acc = _accum(bufs[2 * j + 1], _CB, acc)

            @pl.when(g < _NG - 1)
            def _():
                start(s + _GRP, 1, bufs[2 * j + 1], sems[2 * j + 1])

            out_v[s, pl.ds(0, 16)] = acc[0]
            out_v[s, pl.ds(16, 16)] = acc[1]
            out_v[s, pl.ds(32, 16)] = acc[2]
            out_v[s, pl.ds(48, 16)] = acc[3]
        return carry

    lax.fori_loop(0, _NG, group_body, 0)
    pltpu.sync_copy(out_v, out_hbm.at[pl.ds(wid * _SPW, _SPW)])


def _transpose(tableT, dimidx, tailflat):
    mesh = plsc.VectorSubcoreMesh(core_axis_name="c", subcore_axis_name="s")
    k = functools.partial(
        pl.kernel,
        mesh=mesh,
        compiler_params=pltpu.CompilerParams(needs_layout_passes=False),
        out_type=jax.ShapeDtypeStruct((_V * _D,), jnp.float32),
        scratch_types=[
            pltpu.VMEM((_D,), jnp.int32),
            pltpu.VMEM((_D, _WIN), jnp.float32),
            pltpu.VMEM((_D, _WIN), jnp.float32),
            pltpu.VMEM((_WIN * _D,), jnp.float32),
            pltpu.VMEM((_WIN * _D,), jnp.float32),
            pltpu.SemaphoreType.DMA,
            pltpu.SemaphoreType.DMA,
            pltpu.SemaphoreType.DMA,
            pltpu.SemaphoreType.DMA,
        ],
    )(_tr_body)
    return k(tableT, dimidx, tailflat)


def _pool(ids2, table):
    mesh = plsc.VectorSubcoreMesh(core_axis_name="c", subcore_axis_name="s")
    scratch = [
        pltpu.VMEM((_SPW * _L,), jnp.int32),
        pltpu.VMEM((_SPW, _D), jnp.float32),
    ]
    for _ in range(_GRP):
        scratch.append(pltpu.VMEM((_CA, _D), jnp.float32))
        scratch.append(pltpu.VMEM((_CB, _D), jnp.float32))
    scratch.extend(pltpu.SemaphoreType.DMA for _ in range(2 * _GRP))
    k = functools.partial(
        pl.kernel,
        mesh=mesh,
        compiler_params=pltpu.CompilerParams(use_tc_tiling_on_sc=False),
        out_type=jax.ShapeDtypeStruct((_B, _D), jnp.float32),
        scratch_types=scratch,
    )(_pool_body)
    return k(ids2, table)


def _mlp_body(x_ref, m_ref, w1_ref, b1_ref, w2_ref, b2_ref, out_ref):
    cnt = jnp.sum(m_ref[...], axis=1, keepdims=True) + 1e-9
    x = x_ref[...] / cnt
    h = jnp.dot(x, w1_ref[...], preferred_element_type=jnp.float32) + b1_ref[...]
    h = jnp.maximum(h, 0.0)
    out_ref[...] = jnp.dot(h, w2_ref[...], preferred_element_type=jnp.float32) + b2_ref[...]


def _mlp(pooled, mask, W1, b1, W2, b2):
    blk = 512
    grid = (_B // blk,)
    return pl.pallas_call(
        _mlp_body,
        grid=grid,
        in_specs=[
            pl.BlockSpec((blk, _D), lambda i: (i, 0)),
            pl.BlockSpec((blk, _L), lambda i: (i, 0)),
            pl.BlockSpec((_D, _H), lambda i: (0, 0)),
            pl.BlockSpec((1, _H), lambda i: (0, 0)),
            pl.BlockSpec((_H, _O), lambda i: (0, 0)),
            pl.BlockSpec((1, _O), lambda i: (0, 0)),
        ],
        out_specs=pl.BlockSpec((blk, _O), lambda i: (i, 0)),
        out_shape=jax.ShapeDtypeStruct((_B, _O), jnp.float32),
    )(pooled, mask, W1, b1.reshape(1, _H), W2, b2.reshape(1, _O))


def kernel(input_ids, attention_mask, table, W1, b1, W2, b2):
    ids2 = input_ids.reshape(_NW, _SPW * _L)
    tailflat = table[_NWIN * _WIN:, :].reshape(_TAIL * _D)
    dimidx = jnp.arange(_D, dtype=jnp.int32)
    flat = _transpose(table.T, dimidx, tailflat)
    tab_lin = flat.reshape(_V, _D)
    pooled = _pool(ids2, tab_lin)
    return _mlp(pooled, attention_mask, W1, b1, W2, b2)


# speedup vs baseline: 1.9435x; 1.9435x over previous
import functools

import jax
import jax.numpy as jnp
from jax import lax
from jax.experimental import pallas as pl
from jax.experimental.pallas import tpu as pltpu
from jax.experimental.pallas import tpu_sc as plsc

_V, _D, _H, _O = 1000000, 64, 128, 10
_B, _L = 4096, 200

_NC, _NS = 2, 16
_NW = _NC * _NS
_SPW = _B // _NW
_CA, _CB = 96, 104
_GRP = 4
_NG = _SPW // _GRP


def _accum(buf, nrows, acc):
    def rbody(r, carry):
        a0, a1, a2, a3 = carry
        for u in range(4):
            row = 4 * r + u
            a0 = a0 + buf[row, pl.ds(0, 16)]
            a1 = a1 + buf[row, pl.ds(16, 16)]
            a2 = a2 + buf[row, pl.ds(32, 16)]
            a3 = a3 + buf[row, pl.ds(48, 16)]
        return (a0, a1, a2, a3)
    return lax.fori_loop(0, nrows // 4, rbody, acc)


def _pool_body(ids_hbm, table_hbm, out_hbm, idx_v, out_v, *bufsem):
    bufs = bufsem[:2 * _GRP]
    sems = bufsem[2 * _GRP:]
    wid = lax.axis_index("s") * _NC + lax.axis_index("c")
    pltpu.sync_copy(ids_hbm.at[wid], idx_v)

    def start(s, j, buf, sem):
        if j == 0:
            src = table_hbm.at[idx_v.at[pl.ds(_L * s, _CA)]]
        else:
            src = table_hbm.at[idx_v.at[pl.ds(_L * s + _CA, _CB)]]
        pltpu.make_async_copy(src, buf, sem).start()

    def wait(buf, sem):
        pltpu.make_async_copy(table_hbm.at[idx_v.at[pl.ds(0, buf.shape[0])]],
                              buf, sem).wait()

    for j in range(_GRP):
        start(j, 0, bufs[2 * j], sems[2 * j])
        start(j, 1, bufs[2 * j + 1], sems[2 * j + 1])

    zero = jnp.zeros((16,), jnp.float32)

    def group_body(g, carry):
        for j in range(_GRP):
            s = _GRP * g + j
            wait(bufs[2 * j], sems[2 * j])
            acc = _accum(bufs[2 * j], _CA, (zero, zero, zero, zero))

            @pl.when(g < _NG - 1)
            def _():
                start(s + _GRP, 0, bufs[2 * j], sems[2 * j])

            wait(bufs[2 * j + 1], sems[2 * j + 1])
            acc = _accum(bufs[2 * j + 1], _CB, acc)

            @pl.when(g < _NG - 1)
            def _():
                start(s + _GRP, 1, bufs[2 * j + 1], sems[2 * j + 1])

            out_v[s, pl.ds(0, 16)] = acc[0]
            out_v[s, pl.ds(16, 16)] = acc[1]
            out_v[s, pl.ds(32, 16)] = acc[2]
            out_v[s, pl.ds(48, 16)] = acc[3]
        return carry

    lax.fori_loop(0, _NG, group_body, 0)
    pltpu.sync_copy(out_v, out_hbm.at[pl.ds(wid * _SPW, _SPW)])


def _pool(ids2, table):
    mesh = plsc.VectorSubcoreMesh(core_axis_name="c", subcore_axis_name="s")
    scratch = [
        pltpu.VMEM((_SPW * _L,), jnp.int32),
        pltpu.VMEM((_SPW, _D), jnp.float32),
    ]
    for _ in range(_GRP):
        scratch.append(pltpu.VMEM((_CA, _D), jnp.float32))
        scratch.append(pltpu.VMEM((_CB, _D), jnp.float32))
    scratch.extend(pltpu.SemaphoreType.DMA for _ in range(2 * _GRP))
    k = functools.partial(
        pl.kernel,
        mesh=mesh,
        compiler_params=pltpu.CompilerParams(use_tc_tiling_on_sc=False),
        out_type=jax.ShapeDtypeStruct((_B, _D), jnp.float32),
        scratch_types=scratch,
    )(_pool_body)
    return k(ids2, table)


def _mlp_body(x_ref, m_ref, w1_ref, b1_ref, w2_ref, b2_ref, out_ref):
    cnt = jnp.sum(m_ref[...], axis=1, keepdims=True) + 1e-9
    x = x_ref[...] / cnt
    h = jnp.dot(x, w1_ref[...], preferred_element_type=jnp.float32) + b1_ref[...]
    h = jnp.maximum(h, 0.0)
    out_ref[...] = jnp.dot(h, w2_ref[...], preferred_element_type=jnp.float32) + b2_ref[...]


def _mlp(pooled, mask, W1, b1, W2, b2):
    blk = 512
    grid = (_B // blk,)
    return pl.pallas_call(
        _mlp_body,
        grid=grid,
        in_specs=[
            pl.BlockSpec((blk, _D), lambda i: (i, 0)),
            pl.BlockSpec((blk, _L), lambda i: (i, 0)),
            pl.BlockSpec((_D, _H), lambda i: (0, 0)),
            pl.BlockSpec((1, _H), lambda i: (0, 0)),
            pl.BlockSpec((_H, _O), lambda i: (0, 0)),
            pl.BlockSpec((1, _O), lambda i: (0, 0)),
        ],
        out_specs=pl.BlockSpec((blk, _O), lambda i: (i, 0)),
        out_shape=jax.ShapeDtypeStruct((_B, _O), jnp.float32),
    )(pooled, mask, W1, b1.reshape(1, _H), W2, b2.reshape(1, _O))


def kernel(input_ids, attention_mask, table, W1, b1, W2, b2):
    ids2 = input_ids.reshape(_NW, _SPW * _L)
    pooled = _pool(ids2, table)
    return _mlp(pooled, attention_mask, W1, b1, W2, b2)
